# baseline (speedup 1.0000x reference)
import jax
import jax.numpy as jnp
from jax.experimental import pallas as pl
from jax.experimental.pallas import tpu as pltpu


def _fused_kernel(x_ref, w1_ref, b1_ref, w2_ref, b2_ref, w3_ref, b3_ref,
                  out_ref, xpad_ref, h1pad_ref):
    T = x_ref.shape[1]
    K1 = w1_ref.shape[0]
    K2 = w2_ref.shape[0]
    P1 = K1 // 2
    P2 = K2 // 2

    xpad_ref[0:P1, :] = jnp.zeros_like(xpad_ref[0:P1, :])
    xpad_ref[P1 + T:, :] = jnp.zeros_like(xpad_ref[P1 + T:, :])
    h1pad_ref[0:P2, :] = jnp.zeros_like(h1pad_ref[0:P2, :])
    h1pad_ref[P2 + T:, :] = jnp.zeros_like(h1pad_ref[P2 + T:, :])

    xpad_ref[P1:P1 + T, :] = x_ref[0].astype(jnp.bfloat16)

    acc = None
    for k in range(K1):
        part = jnp.dot(xpad_ref[k:k + T, :], w1_ref[k],
                       preferred_element_type=jnp.float32)
        acc = part if acc is None else acc + part
    h1pad_ref[P2:P2 + T, :] = jnp.maximum(acc + b1_ref[:], 0.0) \
        .astype(jnp.bfloat16)

    acc = None
    for k in range(K2):
        part = jnp.dot(h1pad_ref[k:k + T, :], w2_ref[k],
                       preferred_element_type=jnp.float32)
        acc = part if acc is None else acc + part
    h2 = jnp.maximum(acc + b2_ref[:], 0.0).astype(jnp.bfloat16)

    out_ref[0] = jnp.dot(h2, w3_ref[:], preferred_element_type=jnp.float32) \
        + b3_ref[:]


def kernel(x, W1, b1, W2, b2, W3, b3):
    B, T, C = x.shape
    H, _, K1 = W1.shape
    _, _, K2 = W2.shape
    O = W3.shape[0]

    W1t = jnp.transpose(W1, (2, 1, 0)).astype(jnp.bfloat16)
    W2t = jnp.transpose(W2, (2, 1, 0)).astype(jnp.bfloat16)
    W3t = jnp.transpose(W3[:, :, 0], (1, 0)).astype(jnp.bfloat16)

    out = pl.pallas_call(
        _fused_kernel,
        grid=(B,),
        in_specs=[
            pl.BlockSpec((1, T, C), lambda b: (b, 0, 0)),
            pl.BlockSpec((K1, C, H), lambda b: (0, 0, 0)),
            pl.BlockSpec((1, H), lambda b: (0, 0)),
            pl.BlockSpec((K2, H, H), lambda b: (0, 0, 0)),
            pl.BlockSpec((1, H), lambda b: (0, 0)),
            pl.BlockSpec((H, O), lambda b: (0, 0)),
            pl.BlockSpec((1, O), lambda b: (0, 0)),
        ],
        out_specs=pl.BlockSpec((1, T, O), lambda b: (b, 0, 0)),
        out_shape=jax.ShapeDtypeStruct((B, T, O), jnp.float32),
        scratch_shapes=[
            pltpu.VMEM((T + 2 * (K1 // 2), C), jnp.bfloat16),
            pltpu.VMEM((T + 2 * (K2 // 2), H), jnp.bfloat16),
        ],
        compiler_params=pltpu.CompilerParams(
            dimension_semantics=("parallel",),
        ),
    )(x, W1t, b1[None, :], W2t, b2[None, :], W3t, b3[None, :])
    return out


# speedup vs baseline: 1.1975x; 1.1975x over previous
import jax
import jax.numpy as jnp
from jax.experimental import pallas as pl
from jax.experimental.pallas import tpu as pltpu


def _shift_rows(a, d):
    if d == 0:
        return a
    z = jnp.zeros((abs(d), a.shape[1]), a.dtype)
    if d > 0:
        return jnp.concatenate([a[d:, :], z], axis=0)
    return jnp.concatenate([z, a[:d, :]], axis=0)


def _fused_kernel(x_ref, w1_ref, b1_ref, w2_ref, b2_ref, w3_ref, b3_ref,
                  out_ref, *, H, K1, K2):
    P1, P2 = K1 // 2, K2 // 2
    xb = x_ref[0].astype(jnp.bfloat16)
    z1 = jnp.dot(xb, w1_ref[:], preferred_element_type=jnp.float32)
    acc = b1_ref[:]
    for k in range(K1):
        acc = acc + _shift_rows(z1[:, k * H:(k + 1) * H], k - P1)
    h1 = jnp.maximum(acc, 0.0).astype(jnp.bfloat16)

    z2 = jnp.dot(h1, w2_ref[:], preferred_element_type=jnp.float32)
    acc = b2_ref[:]
    for k in range(K2):
        acc = acc + _shift_rows(z2[:, k * H:(k + 1) * H], k - P2)
    h2 = jnp.maximum(acc, 0.0).astype(jnp.bfloat16)

    out_ref[0] = jnp.dot(h2, w3_ref[:], preferred_element_type=jnp.float32) \
        + b3_ref[:]


def kernel(x, W1, b1, W2, b2, W3, b3):
    import functools
    B, T, C = x.shape
    H, _, K1 = W1.shape
    _, _, K2 = W2.shape
    O = W3.shape[0]

    W1c = jnp.transpose(W1, (1, 2, 0)).reshape(C, K1 * H).astype(jnp.bfloat16)
    W2c = jnp.transpose(W2, (1, 2, 0)).reshape(H, K2 * H).astype(jnp.bfloat16)
    W3t = jnp.transpose(W3[:, :, 0], (1, 0)).astype(jnp.bfloat16)

    out = pl.pallas_call(
        functools.partial(_fused_kernel, H=H, K1=K1, K2=K2),
        grid=(B,),
        in_specs=[
            pl.BlockSpec((1, T, C), lambda b: (b, 0, 0)),
            pl.BlockSpec((C, K1 * H), lambda b: (0, 0)),
            pl.BlockSpec((1, H), lambda b: (0, 0)),
            pl.BlockSpec((H, K2 * H), lambda b: (0, 0)),
            pl.BlockSpec((1, H), lambda b: (0, 0)),
            pl.BlockSpec((H, O), lambda b: (0, 0)),
            pl.BlockSpec((1, O), lambda b: (0, 0)),
        ],
        out_specs=pl.BlockSpec((1, T, O), lambda b: (b, 0, 0)),
        out_shape=jax.ShapeDtypeStruct((B, T, O), jnp.float32),
        compiler_params=pltpu.CompilerParams(
            dimension_semantics=("parallel",),
        ),
    )(x, W1c, b1[None, :], W2c, b2[None, :], W3t, b3[None, :])
    return out


# speedup vs baseline: 1.3425x; 1.1210x over previous
import functools

import jax
import jax.numpy as jnp
from jax.experimental import pallas as pl
from jax.experimental.pallas import tpu as pltpu


def _shift_rows(a, d):
    if d == 0:
        return a
    z = jnp.zeros((abs(d), a.shape[1]), a.dtype)
    if d > 0:
        return jnp.concatenate([a[d:, :], z], axis=0)
    return jnp.concatenate([z, a[:d, :]], axis=0)


def _conv_accum(z, b, H, K, T, G):
    P = K // 2
    outs = []
    for g in range(G):
        zg = z[g * T:(g + 1) * T]
        acc = b
        for k in range(K):
            acc = acc + _shift_rows(zg[:, k * H:(k + 1) * H], k - P)
        outs.append(acc)
    return jnp.concatenate(outs, axis=0) if G > 1 else outs[0]


def _mdot(a, b, nc):
    ch = a.shape[0] // nc
    return jnp.concatenate(
        [jnp.dot(a[i * ch:(i + 1) * ch], b,
                 preferred_element_type=jnp.float32) for i in range(nc)],
        axis=0)

def _fused_kernel(x_ref, w1_ref, b1_ref, w2_ref, b2_ref, w3_ref, b3_ref,
                  out_ref, *, H, K1, K2, G):
    T = x_ref.shape[1]
    O = out_ref.shape[2]
    C = x_ref.shape[2]
    xb = x_ref[:].reshape(G * T, C).astype(jnp.bfloat16)
    z1 = _mdot(xb, w1_ref[:], 4)
    h1 = jnp.maximum(_conv_accum(z1, b1_ref[:], H, K1, T, G),
                     0.0).astype(jnp.bfloat16)
    z2 = _mdot(h1, w2_ref[:], 4)
    h2 = jnp.maximum(_conv_accum(z2, b2_ref[:], H, K2, T, G),
                     0.0).astype(jnp.bfloat16)
    z3 = _mdot(h2, w3_ref[:], 4) + b3_ref[:]
    out_ref[:] = z3.reshape(G, T, O)


def kernel(x, W1, b1, W2, b2, W3, b3):
    B, T, C = x.shape
    H, _, K1 = W1.shape
    _, _, K2 = W2.shape
    O = W3.shape[0]
    G = 2

    W1c = jnp.transpose(W1, (1, 2, 0)).reshape(C, K1 * H).astype(jnp.bfloat16)
    W2c = jnp.transpose(W2, (1, 2, 0)).reshape(H, K2 * H).astype(jnp.bfloat16)
    W3t = jnp.transpose(W3[:, :, 0], (1, 0)).astype(jnp.bfloat16)

    out = pl.pallas_call(
        functools.partial(_fused_kernel, H=H, K1=K1, K2=K2, G=G),
        grid=(B // G,),
        in_specs=[
            pl.BlockSpec((G, T, C), lambda b: (b, 0, 0)),
            pl.BlockSpec((C, K1 * H), lambda b: (0, 0)),
            pl.BlockSpec((1, H), lambda b: (0, 0)),
            pl.BlockSpec((H, K2 * H), lambda b: (0, 0)),
            pl.BlockSpec((1, H), lambda b: (0, 0)),
            pl.BlockSpec((H, O), lambda b: (0, 0)),
            pl.BlockSpec((1, O), lambda b: (0, 0)),
        ],
        out_specs=pl.BlockSpec((G, T, O), lambda b: (b, 0, 0)),
        out_shape=jax.ShapeDtypeStruct((B, T, O), jnp.float32),
        compiler_params=pltpu.CompilerParams(
            dimension_semantics=("parallel",),
        ),
    )(x, W1c, b1[None, :], W2c, b2[None, :], W3t, b3[None, :])
    return out
